# Initial kernel scaffold; baseline (speedup 1.0000x reference)
import jax
import jax.numpy as jnp
from jax.experimental import pallas as pl


def kernel(x, w1, b1, w2, b2):
    raise NotImplementedError("write your pallas kernel here")


# speedup vs baseline: 1.1099x; 1.1099x over previous
import jax
import jax.numpy as jnp
from jax.experimental import pallas as pl
from jax.experimental.pallas import tpu as pltpu

_VMEM_LIMIT = int((64 << 20) * 0.9)


def _ffn_fused_kernel(x_ref, w1_ref, b1_ref, w2_ref, b2_ref, o_ref):
    xb = x_ref[...].astype(jnp.bfloat16)
    h = jnp.dot(xb, w1_ref[...], preferred_element_type=jnp.float32)
    h = jnp.maximum(h + b1_ref[...], 0.0).astype(jnp.bfloat16)
    out = jnp.dot(h, w2_ref[...], preferred_element_type=jnp.float32)
    o_ref[...] = (out + b2_ref[...]).astype(o_ref.dtype)


def kernel(x, w1, b1, w2, b2):
    B, S, D_in = x.shape
    d_in_p, h_p = w1.shape
    d_out_p = w2.shape[1]
    M = B * S

    x2d = x.reshape(M, D_in)

    tm = 512
    grid = (M // tm,)

    out2d = pl.pallas_call(
        _ffn_fused_kernel,
        out_shape=jax.ShapeDtypeStruct((M, d_out_p), x.dtype),
        grid=grid,
        in_specs=[
            pl.BlockSpec((tm, d_in_p), lambda i: (i, 0)),
            pl.BlockSpec((d_in_p, h_p), lambda i: (0, 0)),
            pl.BlockSpec((1, h_p), lambda i: (0, 0)),
            pl.BlockSpec((h_p, d_out_p), lambda i: (0, 0)),
            pl.BlockSpec((1, d_out_p), lambda i: (0, 0)),
        ],
        out_specs=pl.BlockSpec((tm, d_out_p), lambda i: (i, 0)),
        compiler_params=pltpu.CompilerParams(
            dimension_semantics=("parallel",),
            vmem_limit_bytes=_VMEM_LIMIT,
        ),
    )(x2d, w1, b1, w2, b2)
    return out2d.reshape(B, S, d_out_p)
